# baseline (speedup 1.0000x reference)
import math
from functools import partial

import jax
import jax.numpy as jnp
import numpy as np
from jax.experimental import pallas as pl
from jax.experimental.pallas import tpu as pltpu
from jax.experimental.pallas import tpu_sc as plsc

_LMAX = 2
_NCH = 16
_MSG = 8
_RDIM = 8
_DIM = 128
_CUTOFF = 5.0
_NCD = _MSG * _RDIM
_NM = (_LMAX + 1) ** 2
_L_OF_M = [l for l in range(_LMAX + 1) for _ in range(2 * l + 1)]

_WIN = 128
_G0W = 128
_G1W = 256
_EB = 1280
_NB = 200
_GW = 128

_f32 = jnp.float32


def _cg_coef(l1, m1, l2, m2, l3, m3):
    if m1 + m2 != m3 or l3 < abs(l1 - l2) or l3 > l1 + l2:
        return 0.0
    f = math.factorial
    pref = ((2 * l3 + 1) * f(l3 + l1 - l2) * f(l3 - l1 + l2) * f(l1 + l2 - l3)
            / f(l1 + l2 + l3 + 1)) ** 0.5
    pref *= (f(l3 + m3) * f(l3 - m3) * f(l1 - m1) * f(l1 + m1) * f(l2 - m2)
             * f(l2 + m2)) ** 0.5
    s = 0.0
    for k in range(0, l1 + l2 - l3 + 1):
        d = [k, l1 + l2 - l3 - k, l1 - m1 - k, l2 + m2 - k, l3 - l2 + m1 + k,
             l3 - l1 - m2 + k]
        if min(d) < 0:
            continue
        den = 1.0
        for q in d:
            den *= f(q)
        s += (-1) ** k / den
    return pref * s


def _u_mat(l):
    U = np.zeros((2 * l + 1, 2 * l + 1), dtype=complex)
    for m in range(-l, l + 1):
        i = m + l
        if m == 0:
            U[i, l] = 1.0
        elif m > 0:
            U[i, l + m] = (-1) ** m / np.sqrt(2.0)
            U[i, l - m] = 1.0 / np.sqrt(2.0)
        else:
            mm = -m
            U[i, l - mm] = 1j / np.sqrt(2.0)
            U[i, l + mm] = -1j * ((-1) ** mm) / np.sqrt(2.0)
    return U


def _real_cg_block(l1, l2, l3):
    U1, U2, U3 = _u_mat(l1), _u_mat(l2), _u_mat(l3)
    C = np.zeros((2 * l1 + 1, 2 * l2 + 1, 2 * l3 + 1), dtype=complex)
    for a in range(2 * l1 + 1):
        for b in range(2 * l2 + 1):
            for c in range(2 * l3 + 1):
                s = 0.0 + 0.0j
                for m1 in range(-l1, l1 + 1):
                    for m2 in range(-l2, l2 + 1):
                        m3 = m1 + m2
                        if abs(m3) > l3:
                            continue
                        s += (np.conj(U1[a, m1 + l1]) * np.conj(U2[b, m2 + l2])
                              * U3[c, m3 + l3] * _cg_coef(l1, m1, l2, m2, l3, m3))
                C[a, b, c] = s
    return (C.real + C.imag).astype(np.float32)


def _build_cg_groups():
    paths = [(l1, l2, l3)
             for l1 in range(_LMAX + 1) for l2 in range(_LMAX + 1)
             for l3 in range(abs(l1 - l2), min(_LMAX, l1 + l2) + 1)]
    groups = {}
    for p, (l1, l2, l3) in enumerate(paths):
        blk = _real_cg_block(l1, l2, l3)
        for a in range(2 * l1 + 1):
            for b in range(2 * l2 + 1):
                for c in range(2 * l3 + 1):
                    v = float(blk[a, b, c])
                    if abs(v) < 1e-10:
                        continue
                    i, j, k = l1 * l1 + a, l2 * l2 + b, l3 * l3 + c
                    groups.setdefault((i, j), []).append((k, p, v))
    return sorted(groups.items()), len(paths)


_CG_GROUPS, _NPATHS = _build_cg_groups()


def _tp_apply(ViM, HiM, wtp):
    LiM = [None] * _NM
    for (i, j), terms in _CG_GROUPS:
        prod = ViM[i] * HiM[j]
        for (k, p, v) in terms:
            t = prod * (wtp[p:p + 1, :] * np.float32(v))
            LiM[k] = t if LiM[k] is None else LiM[k] + t
    return [x if x is not None else jnp.zeros_like(ViM[0]) for x in LiM]


def _edge_geometry(d, sw, v):
    inv = 1.0 / d
    nvec = ((jax.lax.broadcasted_iota(jnp.int32, (1, _RDIM), 1) + 1)
            .astype(_f32) * np.float32(np.pi / _CUTOFF))
    rb = jnp.sin(d * nvec) * (np.float32(math.sqrt(2.0 / _CUTOFF)) * inv * sw)
    u = v * inv
    x, y, z = u[:, 0:1], u[:, 1:2], u[:, 2:3]
    c1 = np.float32(math.sqrt(3.0))
    c2 = np.float32(math.sqrt(15.0))
    c3 = np.float32(math.sqrt(5.0) / 2.0)
    Y = jnp.concatenate([
        jnp.ones_like(x), c1 * y, c1 * z, c1 * x,
        c2 * x * y, c2 * y * z, c3 * (3.0 * z * z - 1.0), c2 * x * z,
        np.float32(0.5) * c2 * (x * x - y * y)], axis=1)
    return rb, Y


def _xij_cols(mi, rb):
    return jnp.concatenate([mi[:, c:c + 1] * rb for c in range(_MSG)], axis=1)


def _accumulate_sorted(dens_ref, src, contrib, nb):
    iot = jax.lax.broadcasted_iota(jnp.int32, (nb, 1), 0)

    def cond(s):
        return s < nb

    def body(s):
        masked = jnp.where(iot >= s, src, jnp.int32(2 ** 30))
        w0 = (jnp.min(masked) // 8) * 8
        rel = src - w0
        ok = (iot >= s) & (rel < _WIN)
        S = ((rel == jax.lax.broadcasted_iota(jnp.int32, (nb, _WIN), 1))
             & ok).astype(_f32)
        upd = jax.lax.dot_general(S, contrib, (((0,), (0,)), ((), ())),
                                  preferred_element_type=_f32)
        dens_ref[pl.ds(w0, _WIN), :] = dens_ref[pl.ds(w0, _WIN), :] + upd
        return s + jnp.sum(ok.astype(jnp.int32))

    jax.lax.while_loop(cond, body, jnp.int32(0))


def _seg0_kernel(src_ref, d_ref, vec_ref, sw_ref, g_ref, dens_ref):
    @pl.when(pl.program_id(0) == 0)
    def _():
        dens_ref[...] = jnp.zeros(dens_ref.shape, _f32)

    nb = src_ref.shape[0]
    rb, Y = _edge_geometry(d_ref[...], sw_ref[...], vec_ref[...])
    mi = g_ref[...][:, 0:_MSG]
    xij = _xij_cols(mi, rb)
    contrib = jnp.concatenate([xij * Y[:, m:m + 1] for m in range(_NM)], axis=1)
    _accumulate_sorted(dens_ref, src_ref[...], contrib, nb)


def _seg1_kernel(src_ref, d_ref, vec_ref, sw_ref, g_ref, wrhoT_ref, dens_ref):
    @pl.when(pl.program_id(0) == 0)
    def _():
        dens_ref[...] = jnp.zeros(dens_ref.shape, _f32)

    nb = src_ref.shape[0]
    rb, _ = _edge_geometry(d_ref[...], sw_ref[...], vec_ref[...])
    g = g_ref[...]
    mi = g[:, 0:_MSG]
    xij = _xij_cols(mi, rb)
    wrhoT = wrhoT_ref[...]
    pieces = []
    for m in range(_NM):
        Vg_m = g[:, _MSG + m * _NCH:_MSG + (m + 1) * _NCH]
        R_m = jnp.dot(Vg_m, wrhoT[m], preferred_element_type=_f32)
        pieces.append(xij * R_m)
    contrib = jnp.concatenate(pieces, axis=1)
    _accumulate_sorted(dens_ref, src_ref[...], contrib, nb)


def _node_pre_kernel(spec_ref, wspec_ref, bspec_ref, wmsg_ref, xi_ref, mi_ref):
    sp = spec_ref[...]
    nb = sp.shape[0]
    enc = (sp == jax.lax.broadcasted_iota(jnp.int32, (nb, 64), 1)).astype(_f32)
    xi = jnp.dot(enc, wspec_ref[...], preferred_element_type=_f32) + bspec_ref[...]
    xi_ref[...] = xi
    mi = jnp.dot(xi, wmsg_ref[...], preferred_element_type=_f32)
    mi_ref[...] = jnp.concatenate(
        [mi, jnp.zeros((nb, _G0W - _MSG), _f32)], axis=1)


def _node_layer(dens, xi, wviT_or_vi, wdmaT, wdmbT, wtpa, wtpb, wla, bla,
                wlb, blb, from_density):
    if from_density:
        wviT = wviT_or_vi
        ViM = [jnp.dot(dens[:, m * _NCD:(m + 1) * _NCD], wviT[m],
                       preferred_element_type=_f32) for m in range(_NM)]
    else:
        vi = wviT_or_vi
        ViM = [vi[:, m * _NCH:(m + 1) * _NCH] for m in range(_NM)]
    HiaM = [jnp.dot(dens[:, m * _NCD:(m + 1) * _NCD], wdmaT,
                    preferred_element_type=_f32) for m in range(_NM)]
    LiaM = _tp_apply(ViM, HiaM, wtpa)
    ViM = [ViM[m] + LiaM[m] for m in range(_NM)]
    HibM = [jnp.dot(dens[:, m * _NCD:(m + 1) * _NCD], wdmbT,
                    preferred_element_type=_f32) for m in range(_NM)]
    LibM = _tp_apply(ViM, HibM, wtpb)
    ViM = [ViM[m] + LibM[m] for m in range(_NM)]
    h = jnp.concatenate([xi, dens[:, 0:_NCD], LiaM[0], LibM[0]], axis=1)
    pre = jnp.dot(h, wla, preferred_element_type=_f32) + bla
    act = pre * jax.nn.sigmoid(pre)
    dxi = jnp.dot(act, wlb, preferred_element_type=_f32) + blb
    return xi + dxi, ViM


def _node_mid_kernel(dens_ref, xi_ref, wviT_ref, wdm0T_ref, wdm1T_ref,
                     wtp0_ref, wtp1_ref, wla_ref, bla_ref, wlb_ref, blb_ref,
                     wmsg_ref, xi1_ref, vi_ref, t1_ref):
    xi1, ViM = _node_layer(
        dens_ref[...], xi_ref[...], wviT_ref[...], wdm0T_ref[...],
        wdm1T_ref[...], wtp0_ref[...], wtp1_ref[...], wla_ref[...],
        bla_ref[...], wlb_ref[...], blb_ref[...], from_density=True)
    xi1_ref[...] = xi1
    vi = jnp.concatenate(ViM, axis=1)
    vi_ref[...] = vi
    mi1 = jnp.dot(xi1, wmsg_ref[...], preferred_element_type=_f32)
    pad = _G1W - _MSG - _NCH * _NM
    t1_ref[...] = jnp.concatenate(
        [mi1, vi, jnp.zeros((mi1.shape[0], pad), _f32)], axis=1)


def _node_fin_kernel(d0_ref, d1_ref, xi_ref, vi_ref, wdm0T_ref, wdm1T_ref,
                     wtp0_ref, wtp1_ref, wla_ref, bla_ref, wlb_ref, blb_ref,
                     xiF_ref, viF_ref):
    dens = d0_ref[...] + d1_ref[...]
    xiF, ViM = _node_layer(
        dens, xi_ref[...], vi_ref[...], wdm0T_ref[...], wdm1T_ref[...],
        wtp0_ref[...], wtp1_ref[...], wla_ref[...], bla_ref[...],
        wlb_ref[...], blb_ref[...], from_density=False)
    xiF_ref[...] = xiF
    viF_ref[...] = jnp.concatenate(ViM, axis=1)


def _sc_gather(table, idx):
    n_idx = idx.shape[0]
    width = table.shape[1]
    idx2 = idx.reshape(1, n_idx)
    mesh = plsc.VectorSubcoreMesh(core_axis_name="c", subcore_axis_name="s")

    @partial(pl.kernel,
             out_type=jax.ShapeDtypeStruct((n_idx, width), table.dtype),
             mesh=mesh)
    def gk(tab_hbm, i_hbm, o_hbm):
        def body(i_vmem, o_vmem):
            pltpu.sync_copy(tab_hbm.at[i_vmem.at[0]], o_vmem)

        pltpu.emit_pipeline(
            body,
            grid=(n_idx // _GW,),
            in_specs=[pl.BlockSpec((1, _GW), lambda i: (0, i))],
            out_specs=[pl.BlockSpec((_GW, width), lambda i: (i, 0))],
            core_axis_name=("c", "s"),
            dimension_semantics=(pltpu.PARALLEL,),
        )(i_hbm, o_hbm)

    return gk(table, idx2)


def _tc_params(vmem_mb, parallel=False):
    sem = ("parallel",) if parallel else ("arbitrary",)
    return pltpu.CompilerParams(dimension_semantics=sem,
                                vmem_limit_bytes=vmem_mb * 1024 * 1024)


def kernel(species, edge_src, edge_dst, distances, vec, switch,
           W_spec, b_spec, W_msg0, b_msg0, W_msg1, b_msg1, W_vi, W_rho,
           W_dm00, W_dm01, W_dm10, W_dm11, W_tp00, W_tp01, W_tp10, W_tp11,
           W_lat0a, b_lat0a, W_lat0b, b_lat0b, W_lat1a, b_lat1a, W_lat1b,
           b_lat1b):
    N = species.shape[0]
    E = edge_src.shape[0]
    n_pad = ((N + _WIN + 7) // 8) * 8
    e_pad = ((E + _EB - 1) // _EB) * _EB
    nb = _NB if N % _NB == 0 else N
    n_grid = N // nb

    src2 = edge_src.astype(jnp.int32).reshape(E, 1)
    dst1 = edge_dst.astype(jnp.int32)
    d2 = distances.astype(_f32).reshape(E, 1)
    vec2 = vec.astype(_f32)
    sw2 = switch.astype(_f32).reshape(E, 1)
    if e_pad != E:
        p = e_pad - E
        src2 = jnp.concatenate([src2, jnp.full((p, 1), N, jnp.int32)])
        dst1 = jnp.concatenate([dst1, jnp.zeros((p,), jnp.int32)])
        d2 = jnp.concatenate([d2, jnp.ones((p, 1), _f32)])
        vec2 = jnp.concatenate([vec2, jnp.ones((p, 3), _f32)])
        sw2 = jnp.concatenate([sw2, jnp.zeros((p, 1), _f32)])

    lom = np.asarray(_L_OF_M)
    wspec_p = jnp.concatenate(
        [W_spec, jnp.zeros((64 - W_spec.shape[0], _DIM), _f32)], axis=0)
    bspec2 = b_spec.reshape(1, _DIM)
    wviT = jnp.transpose(W_vi[lom], (0, 2, 1))
    wrhoT = jnp.transpose(W_rho[lom], (0, 2, 1))
    wdm00T, wdm01T = W_dm00.T, W_dm01.T
    wdm10T, wdm11T = W_dm10.T, W_dm11.T
    bl0a, bl0b = b_lat0a.reshape(1, -1), b_lat0b.reshape(1, -1)
    bl1a, bl1b = b_lat1a.reshape(1, -1), b_lat1b.reshape(1, -1)

    ebs = lambda w: pl.BlockSpec((_EB, w), lambda i: (i, 0))
    nbs = lambda w: pl.BlockSpec((nb, w), lambda i: (i, 0))
    full = lambda *s: pl.BlockSpec(s, lambda i: tuple(0 for _ in s))
    dspec = pl.BlockSpec((n_pad, _NCD * _NM), lambda i: (0, 0))

    xi0, mi0p = pl.pallas_call(
        _node_pre_kernel,
        grid=(n_grid,),
        in_specs=[nbs(1), full(64, _DIM), full(1, _DIM), full(_DIM, _MSG)],
        out_specs=[nbs(_DIM), nbs(_G0W)],
        out_shape=[jax.ShapeDtypeStruct((N, _DIM), _f32),
                   jax.ShapeDtypeStruct((N, _G0W), _f32)],
        compiler_params=_tc_params(64, parallel=True),
    )(species.astype(jnp.int32).reshape(N, 1), wspec_p, bspec2, W_msg0)

    g0 = _sc_gather(mi0p, dst1)

    dens0 = pl.pallas_call(
        _seg0_kernel,
        grid=(e_pad // _EB,),
        in_specs=[ebs(1), ebs(1), ebs(3), ebs(1), ebs(_G0W)],
        out_specs=dspec,
        out_shape=jax.ShapeDtypeStruct((n_pad, _NCD * _NM), _f32),
        compiler_params=_tc_params(56),
    )(src2, d2, vec2, sw2, g0)

    xi1, vi1, t1 = pl.pallas_call(
        _node_mid_kernel,
        grid=(n_grid,),
        in_specs=[nbs(_NCD * _NM), nbs(_DIM), full(_NM, _NCD, _NCH),
                  full(_NCD, _NCH), full(_NCD, _NCH),
                  full(_NPATHS, _NCH), full(_NPATHS, _NCH),
                  full(_DIM + _NCD + 2 * _NCH, _DIM), full(1, _DIM),
                  full(_DIM, _DIM), full(1, _DIM), full(_DIM, _MSG)],
        out_specs=[nbs(_DIM), nbs(_NCH * _NM), nbs(_G1W)],
        out_shape=[jax.ShapeDtypeStruct((N, _DIM), _f32),
                   jax.ShapeDtypeStruct((N, _NCH * _NM), _f32),
                   jax.ShapeDtypeStruct((N, _G1W), _f32)],
        compiler_params=_tc_params(64, parallel=True),
    )(dens0, xi0, wviT, wdm00T, wdm01T, W_tp00, W_tp01,
      W_lat0a, bl0a, W_lat0b, bl0b, W_msg1)

    g1 = _sc_gather(t1, dst1)

    dens1 = pl.pallas_call(
        _seg1_kernel,
        grid=(e_pad // _EB,),
        in_specs=[ebs(1), ebs(1), ebs(3), ebs(1), ebs(_G1W),
                  full(_NM, _NCH, _NCD)],
        out_specs=dspec,
        out_shape=jax.ShapeDtypeStruct((n_pad, _NCD * _NM), _f32),
        compiler_params=_tc_params(56),
    )(src2, d2, vec2, sw2, g1, wrhoT)

    xiF, viF = pl.pallas_call(
        _node_fin_kernel,
        grid=(n_grid,),
        in_specs=[nbs(_NCD * _NM), nbs(_NCD * _NM), nbs(_DIM),
                  nbs(_NCH * _NM), full(_NCD, _NCH), full(_NCD, _NCH),
                  full(_NPATHS, _NCH), full(_NPATHS, _NCH),
                  full(_DIM + _NCD + 2 * _NCH, _DIM), full(1, _DIM),
                  full(_DIM, _DIM), full(1, _DIM)],
        out_specs=[nbs(_DIM), nbs(_NCH * _NM)],
        out_shape=[jax.ShapeDtypeStruct((N, _DIM), _f32),
                   jax.ShapeDtypeStruct((N, _NCH * _NM), _f32)],
        compiler_params=_tc_params(64, parallel=True),
    )(dens0, dens1, xi1, vi1, wdm10T, wdm11T, W_tp10, W_tp11,
      W_lat1a, bl1a, W_lat1b, bl1b)

    Vi_out = viF.reshape(N, _NM, _NCH).transpose(0, 2, 1)
    return xiF, Vi_out


# speedup vs baseline: 1.8778x; 1.8778x over previous
import math
from functools import partial

import jax
import jax.numpy as jnp
import numpy as np
from jax.experimental import pallas as pl
from jax.experimental.pallas import tpu as pltpu
from jax.experimental.pallas import tpu_sc as plsc

_LMAX = 2
_NCH = 16
_MSG = 8
_RDIM = 8
_DIM = 128
_CUTOFF = 5.0
_NCD = _MSG * _RDIM
_NM = (_LMAX + 1) ** 2
_L_OF_M = [l for l in range(_LMAX + 1) for _ in range(2 * l + 1)]

_WIN = 128
_G0W = 128
_G1W = 256
_EB = 1280
_NB = 200
_GW = 128

_f32 = jnp.float32


def _cg_coef(l1, m1, l2, m2, l3, m3):
    if m1 + m2 != m3 or l3 < abs(l1 - l2) or l3 > l1 + l2:
        return 0.0
    f = math.factorial
    pref = ((2 * l3 + 1) * f(l3 + l1 - l2) * f(l3 - l1 + l2) * f(l1 + l2 - l3)
            / f(l1 + l2 + l3 + 1)) ** 0.5
    pref *= (f(l3 + m3) * f(l3 - m3) * f(l1 - m1) * f(l1 + m1) * f(l2 - m2)
             * f(l2 + m2)) ** 0.5
    s = 0.0
    for k in range(0, l1 + l2 - l3 + 1):
        d = [k, l1 + l2 - l3 - k, l1 - m1 - k, l2 + m2 - k, l3 - l2 + m1 + k,
             l3 - l1 - m2 + k]
        if min(d) < 0:
            continue
        den = 1.0
        for q in d:
            den *= f(q)
        s += (-1) ** k / den
    return pref * s


def _u_mat(l):
    U = np.zeros((2 * l + 1, 2 * l + 1), dtype=complex)
    for m in range(-l, l + 1):
        i = m + l
        if m == 0:
            U[i, l] = 1.0
        elif m > 0:
            U[i, l + m] = (-1) ** m / np.sqrt(2.0)
            U[i, l - m] = 1.0 / np.sqrt(2.0)
        else:
            mm = -m
            U[i, l - mm] = 1j / np.sqrt(2.0)
            U[i, l + mm] = -1j * ((-1) ** mm) / np.sqrt(2.0)
    return U


def _real_cg_block(l1, l2, l3):
    U1, U2, U3 = _u_mat(l1), _u_mat(l2), _u_mat(l3)
    C = np.zeros((2 * l1 + 1, 2 * l2 + 1, 2 * l3 + 1), dtype=complex)
    for a in range(2 * l1 + 1):
        for b in range(2 * l2 + 1):
            for c in range(2 * l3 + 1):
                s = 0.0 + 0.0j
                for m1 in range(-l1, l1 + 1):
                    for m2 in range(-l2, l2 + 1):
                        m3 = m1 + m2
                        if abs(m3) > l3:
                            continue
                        s += (np.conj(U1[a, m1 + l1]) * np.conj(U2[b, m2 + l2])
                              * U3[c, m3 + l3] * _cg_coef(l1, m1, l2, m2, l3, m3))
                C[a, b, c] = s
    return (C.real + C.imag).astype(np.float32)


def _build_cg_groups():
    paths = [(l1, l2, l3)
             for l1 in range(_LMAX + 1) for l2 in range(_LMAX + 1)
             for l3 in range(abs(l1 - l2), min(_LMAX, l1 + l2) + 1)]
    groups = {}
    for p, (l1, l2, l3) in enumerate(paths):
        blk = _real_cg_block(l1, l2, l3)
        for a in range(2 * l1 + 1):
            for b in range(2 * l2 + 1):
                for c in range(2 * l3 + 1):
                    v = float(blk[a, b, c])
                    if abs(v) < 1e-10:
                        continue
                    i, j, k = l1 * l1 + a, l2 * l2 + b, l3 * l3 + c
                    groups.setdefault((i, j), []).append((k, p, v))
    return sorted(groups.items()), len(paths)


_CG_GROUPS, _NPATHS = _build_cg_groups()
_NGRP = len(_CG_GROUPS)
_GX = _NGRP * _NCH


def _build_tp_tables():
    EV = np.zeros((_NM * _NCH, _GX), np.float32)
    EH = np.zeros((_NM * _NCH, _GX), np.float32)
    B = np.zeros((_NPATHS, _GX, _NM * _NCH), np.float32)
    for g, ((i, j), terms) in enumerate(_CG_GROUPS):
        for c in range(_NCH):
            EV[i * _NCH + c, g * _NCH + c] = 1.0
            EH[j * _NCH + c, g * _NCH + c] = 1.0
        for (k, p, v) in terms:
            for c in range(_NCH):
                B[p, g * _NCH + c, k * _NCH + c] += v
    return EV, EH, B


_EV_NP, _EH_NP, _B_NP = _build_tp_tables()
_LANE16 = np.tile(np.arange(_NCH), _NM)


def _rep_mats():
    repM = np.zeros((_MSG, _NCD * _NM), np.float32)
    repR = np.zeros((_RDIM, _NCD * _NM), np.float32)
    repY = np.zeros((_NM, _NCD * _NM), np.float32)
    for m in range(_NM):
        for c in range(_MSG):
            for r in range(_RDIM):
                col = m * _NCD + c * _RDIM + r
                repM[c, col] = 1.0
                repR[r, col] = 1.0
                repY[m, col] = 1.0
    return repM, repR, repY


_REPM_NP, _REPR_NP, _REPY_NP = _rep_mats()


def _edge_geometry(d, sw, v):
    inv = 1.0 / d
    nvec = ((jax.lax.broadcasted_iota(jnp.int32, (1, _RDIM), 1) + 1)
            .astype(_f32) * np.float32(np.pi / _CUTOFF))
    rb = jnp.sin(d * nvec) * (np.float32(math.sqrt(2.0 / _CUTOFF)) * inv * sw)
    u = v * inv
    x, y, z = u[:, 0:1], u[:, 1:2], u[:, 2:3]
    c1 = np.float32(math.sqrt(3.0))
    c2 = np.float32(math.sqrt(15.0))
    c3 = np.float32(math.sqrt(5.0) / 2.0)
    Y = jnp.concatenate([
        jnp.ones_like(x), c1 * y, c1 * z, c1 * x,
        c2 * x * y, c2 * y * z, c3 * (3.0 * z * z - 1.0), c2 * x * z,
        np.float32(0.5) * c2 * (x * x - y * y)], axis=1)
    return rb, Y


def _mm(a, b):
    return jnp.dot(a, b, preferred_element_type=_f32)


def _accumulate_sorted(dens_ref, src, contrib, nb):
    iot = jax.lax.broadcasted_iota(jnp.int32, (nb, 1), 0)

    def cond(s):
        return s < nb

    def body(s):
        masked = jnp.where(iot >= s, src, jnp.int32(2 ** 30))
        w0 = (jnp.min(masked) // 8) * 8
        rel = src - w0
        ok = (iot >= s) & (rel < _WIN)
        S = ((rel == jax.lax.broadcasted_iota(jnp.int32, (nb, _WIN), 1))
             & ok).astype(_f32)
        upd = jax.lax.dot_general(S, contrib, (((0,), (0,)), ((), ())),
                                  preferred_element_type=_f32)
        dens_ref[pl.ds(w0, _WIN), :] = dens_ref[pl.ds(w0, _WIN), :] + upd
        return s + jnp.sum(ok.astype(jnp.int32))

    jax.lax.while_loop(cond, body, jnp.int32(0))


def _seg0_kernel(src_ref, d_ref, vec_ref, sw_ref, g_ref, repM_ref, repR_ref,
                 repY_ref, dens_ref):
    @pl.when(pl.program_id(0) == 0)
    def _():
        dens_ref[...] = jnp.zeros(dens_ref.shape, _f32)

    nb = src_ref.shape[0]
    rb, Y = _edge_geometry(d_ref[...], sw_ref[...], vec_ref[...])
    mi = g_ref[...][:, 0:_MSG]
    contrib = (_mm(Y, repY_ref[...]) * _mm(mi, repM_ref[...])
               * _mm(rb, repR_ref[...]))
    _accumulate_sorted(dens_ref, src_ref[...], contrib, nb)


def _seg1_kernel(src_ref, d_ref, vec_ref, sw_ref, g_ref, wrho_bd_ref,
                 repM_ref, repR_ref, dens_ref):
    @pl.when(pl.program_id(0) == 0)
    def _():
        dens_ref[...] = jnp.zeros(dens_ref.shape, _f32)

    nb = src_ref.shape[0]
    rb, _ = _edge_geometry(d_ref[...], sw_ref[...], vec_ref[...])
    g = g_ref[...]
    mi = g[:, 0:_MSG]
    Rfull = _mm(g[:, _MSG:_MSG + _NCH * _NM], wrho_bd_ref[...])
    contrib = _mm(mi, repM_ref[...]) * _mm(rb, repR_ref[...]) * Rfull
    _accumulate_sorted(dens_ref, src_ref[...], contrib, nb)


def _node_pre_kernel(spec_ref, wspec_ref, bspec_ref, wmsg_ref, xi_ref, mi_ref):
    sp = spec_ref[...]
    nb = sp.shape[0]
    enc = (sp == jax.lax.broadcasted_iota(jnp.int32, (nb, 64), 1)).astype(_f32)
    xi = jnp.dot(enc, wspec_ref[...], preferred_element_type=_f32) + bspec_ref[...]
    xi_ref[...] = xi
    mi = jnp.dot(xi, wmsg_ref[...], preferred_element_type=_f32)
    mi_ref[...] = jnp.concatenate(
        [mi, jnp.zeros((nb, _G0W - _MSG), _f32)], axis=1)


def _node_layer(dens, xi, vi, wdma_bd, wdmb_bd, EV, EH, Ma, Mb,
                wla, bla, wlb, blb):
    Hia = _mm(dens, wdma_bd)
    Lia = _mm(_mm(vi, EV) * _mm(Hia, EH), Ma)
    vi = vi + Lia
    Hib = _mm(dens, wdmb_bd)
    Lib = _mm(_mm(vi, EV) * _mm(Hib, EH), Mb)
    vi = vi + Lib
    h = jnp.concatenate(
        [xi, dens[:, 0:_NCD], Lia[:, 0:_NCH], Lib[:, 0:_NCH]], axis=1)
    pre = _mm(h, wla) + bla
    act = pre * jax.nn.sigmoid(pre)
    return xi + _mm(act, wlb) + blb, vi


def _node_mid_kernel(dens_ref, xi_ref, wvibd_ref, wdma_ref, wdmb_ref,
                     ev_ref, eh_ref, ma_ref, mb_ref, wla_ref, bla_ref,
                     wlb_ref, blb_ref, wmsg_ref, xi1_ref, vi_ref, t1_ref):
    dens = dens_ref[...]
    vi0 = _mm(dens, wvibd_ref[...])
    xi1, vi = _node_layer(
        dens, xi_ref[...], vi0, wdma_ref[...], wdmb_ref[...], ev_ref[...],
        eh_ref[...], ma_ref[...], mb_ref[...], wla_ref[...], bla_ref[...],
        wlb_ref[...], blb_ref[...])
    xi1_ref[...] = xi1
    vi_ref[...] = vi
    mi1 = _mm(xi1, wmsg_ref[...])
    pad = _G1W - _MSG - _NCH * _NM
    t1_ref[...] = jnp.concatenate(
        [mi1, vi, jnp.zeros((mi1.shape[0], pad), _f32)], axis=1)


def _node_fin_kernel(d0_ref, d1_ref, xi_ref, vi_ref, wdma_ref, wdmb_ref,
                     ev_ref, eh_ref, ma_ref, mb_ref, wla_ref, bla_ref,
                     wlb_ref, blb_ref, xiF_ref, viF_ref):
    dens = d0_ref[...] + d1_ref[...]
    xiF, vi = _node_layer(
        dens, xi_ref[...], vi_ref[...], wdma_ref[...], wdmb_ref[...],
        ev_ref[...], eh_ref[...], ma_ref[...], mb_ref[...], wla_ref[...],
        bla_ref[...], wlb_ref[...], blb_ref[...])
    xiF_ref[...] = xiF
    viF_ref[...] = vi


def _sc_gather(table, idx):
    n_idx = idx.shape[0]
    width = table.shape[1]
    idx2 = idx.reshape(1, n_idx)
    mesh = plsc.VectorSubcoreMesh(core_axis_name="c", subcore_axis_name="s")

    @partial(pl.kernel,
             out_type=jax.ShapeDtypeStruct((n_idx, width), table.dtype),
             mesh=mesh)
    def gk(tab_hbm, i_hbm, o_hbm):
        def body(i_vmem, o_vmem):
            pltpu.sync_copy(tab_hbm.at[i_vmem.at[0]], o_vmem)

        pltpu.emit_pipeline(
            body,
            grid=(n_idx // _GW,),
            in_specs=[pl.BlockSpec((1, _GW), lambda i: (0, i))],
            out_specs=[pl.BlockSpec((_GW, width), lambda i: (i, 0))],
            core_axis_name=("c", "s"),
            dimension_semantics=(pltpu.PARALLEL,),
        )(i_hbm, o_hbm)

    return gk(table, idx2)


def _tc_params(vmem_mb, parallel=False):
    sem = ("parallel",) if parallel else ("arbitrary",)
    return pltpu.CompilerParams(dimension_semantics=sem,
                                vmem_limit_bytes=vmem_mb * 1024 * 1024)


def kernel(species, edge_src, edge_dst, distances, vec, switch,
           W_spec, b_spec, W_msg0, b_msg0, W_msg1, b_msg1, W_vi, W_rho,
           W_dm00, W_dm01, W_dm10, W_dm11, W_tp00, W_tp01, W_tp10, W_tp11,
           W_lat0a, b_lat0a, W_lat0b, b_lat0b, W_lat1a, b_lat1a, W_lat1b,
           b_lat1b):
    N = species.shape[0]
    E = edge_src.shape[0]
    n_pad = ((N + _WIN + 7) // 8) * 8
    e_pad = ((E + _EB - 1) // _EB) * _EB
    nb = _NB if N % _NB == 0 else N
    n_grid = N // nb

    src2 = edge_src.astype(jnp.int32).reshape(E, 1)
    dst1 = edge_dst.astype(jnp.int32)
    d2 = distances.astype(_f32).reshape(E, 1)
    vec2 = vec.astype(_f32)
    sw2 = switch.astype(_f32).reshape(E, 1)
    if e_pad != E:
        p = e_pad - E
        src2 = jnp.concatenate([src2, jnp.full((p, 1), N, jnp.int32)])
        dst1 = jnp.concatenate([dst1, jnp.zeros((p,), jnp.int32)])
        d2 = jnp.concatenate([d2, jnp.ones((p, 1), _f32)])
        vec2 = jnp.concatenate([vec2, jnp.ones((p, 3), _f32)])
        sw2 = jnp.concatenate([sw2, jnp.zeros((p, 1), _f32)])

    lom = np.asarray(_L_OF_M)
    wspec_p = jnp.concatenate(
        [W_spec, jnp.zeros((64 - W_spec.shape[0], _DIM), _f32)], axis=0)
    bspec2 = b_spec.reshape(1, _DIM)
    wviT = jnp.transpose(W_vi[lom], (0, 2, 1))
    wrhoT = jnp.transpose(W_rho[lom], (0, 2, 1))
    bd = jax.scipy.linalg.block_diag
    wvi_bd = bd(*[wviT[m] for m in range(_NM)])
    wrho_bd = bd(*[wrhoT[m] for m in range(_NM)])
    wdm00_bd = bd(*([W_dm00.T] * _NM))
    wdm01_bd = bd(*([W_dm01.T] * _NM))
    wdm10_bd = bd(*([W_dm10.T] * _NM))
    wdm11_bd = bd(*([W_dm11.T] * _NM))
    Bt = jnp.asarray(_B_NP)
    Ma0 = jnp.einsum('pgk,pk->gk', Bt, W_tp00[:, _LANE16])
    Mb0 = jnp.einsum('pgk,pk->gk', Bt, W_tp01[:, _LANE16])
    Ma1 = jnp.einsum('pgk,pk->gk', Bt, W_tp10[:, _LANE16])
    Mb1 = jnp.einsum('pgk,pk->gk', Bt, W_tp11[:, _LANE16])
    EV, EH = jnp.asarray(_EV_NP), jnp.asarray(_EH_NP)
    repM, repR, repY = (jnp.asarray(_REPM_NP), jnp.asarray(_REPR_NP),
                        jnp.asarray(_REPY_NP))
    bl0a, bl0b = b_lat0a.reshape(1, -1), b_lat0b.reshape(1, -1)
    bl1a, bl1b = b_lat1a.reshape(1, -1), b_lat1b.reshape(1, -1)

    ebs = lambda w: pl.BlockSpec((_EB, w), lambda i: (i, 0))
    nbs = lambda w: pl.BlockSpec((nb, w), lambda i: (i, 0))
    full = lambda *s: pl.BlockSpec(s, lambda i: tuple(0 for _ in s))
    dspec = pl.BlockSpec((n_pad, _NCD * _NM), lambda i: (0, 0))

    xi0, mi0p = pl.pallas_call(
        _node_pre_kernel,
        grid=(n_grid,),
        in_specs=[nbs(1), full(64, _DIM), full(1, _DIM), full(_DIM, _MSG)],
        out_specs=[nbs(_DIM), nbs(_G0W)],
        out_shape=[jax.ShapeDtypeStruct((N, _DIM), _f32),
                   jax.ShapeDtypeStruct((N, _G0W), _f32)],
        compiler_params=_tc_params(64, parallel=True),
    )(species.astype(jnp.int32).reshape(N, 1), wspec_p, bspec2, W_msg0)

    g0 = _sc_gather(mi0p, dst1)

    dens0 = pl.pallas_call(
        _seg0_kernel,
        grid=(e_pad // _EB,),
        in_specs=[ebs(1), ebs(1), ebs(3), ebs(1), ebs(_G0W),
                  full(_MSG, _NCD * _NM), full(_RDIM, _NCD * _NM),
                  full(_NM, _NCD * _NM)],
        out_specs=dspec,
        out_shape=jax.ShapeDtypeStruct((n_pad, _NCD * _NM), _f32),
        compiler_params=_tc_params(56),
    )(src2, d2, vec2, sw2, g0, repM, repR, repY)

    xi1, vi1, t1 = pl.pallas_call(
        _node_mid_kernel,
        grid=(n_grid,),
        in_specs=[nbs(_NCD * _NM), nbs(_DIM),
                  full(_NCD * _NM, _NCH * _NM), full(_NCD * _NM, _NCH * _NM),
                  full(_NCD * _NM, _NCH * _NM),
                  full(_NCH * _NM, _GX), full(_NCH * _NM, _GX),
                  full(_GX, _NCH * _NM), full(_GX, _NCH * _NM),
                  full(_DIM + _NCD + 2 * _NCH, _DIM), full(1, _DIM),
                  full(_DIM, _DIM), full(1, _DIM), full(_DIM, _MSG)],
        out_specs=[nbs(_DIM), nbs(_NCH * _NM), nbs(_G1W)],
        out_shape=[jax.ShapeDtypeStruct((N, _DIM), _f32),
                   jax.ShapeDtypeStruct((N, _NCH * _NM), _f32),
                   jax.ShapeDtypeStruct((N, _G1W), _f32)],
        compiler_params=_tc_params(64, parallel=True),
    )(dens0, xi0, wvi_bd, wdm00_bd, wdm01_bd, EV, EH, Ma0, Mb0,
      W_lat0a, bl0a, W_lat0b, bl0b, W_msg1)

    g1 = _sc_gather(t1, dst1)

    dens1 = pl.pallas_call(
        _seg1_kernel,
        grid=(e_pad // _EB,),
        in_specs=[ebs(1), ebs(1), ebs(3), ebs(1), ebs(_G1W),
                  full(_NCH * _NM, _NCD * _NM), full(_MSG, _NCD * _NM),
                  full(_RDIM, _NCD * _NM)],
        out_specs=dspec,
        out_shape=jax.ShapeDtypeStruct((n_pad, _NCD * _NM), _f32),
        compiler_params=_tc_params(56),
    )(src2, d2, vec2, sw2, g1, wrho_bd, repM, repR)

    xiF, viF = pl.pallas_call(
        _node_fin_kernel,
        grid=(n_grid,),
        in_specs=[nbs(_NCD * _NM), nbs(_NCD * _NM), nbs(_DIM),
                  nbs(_NCH * _NM),
                  full(_NCD * _NM, _NCH * _NM), full(_NCD * _NM, _NCH * _NM),
                  full(_NCH * _NM, _GX), full(_NCH * _NM, _GX),
                  full(_GX, _NCH * _NM), full(_GX, _NCH * _NM),
                  full(_DIM + _NCD + 2 * _NCH, _DIM), full(1, _DIM),
                  full(_DIM, _DIM), full(1, _DIM)],
        out_specs=[nbs(_DIM), nbs(_NCH * _NM)],
        out_shape=[jax.ShapeDtypeStruct((N, _DIM), _f32),
                   jax.ShapeDtypeStruct((N, _NCH * _NM), _f32)],
        compiler_params=_tc_params(64, parallel=True),
    )(dens0, dens1, xi1, vi1, wdm10_bd, wdm11_bd, EV, EH, Ma1, Mb1,
      W_lat1a, bl1a, W_lat1b, bl1b)

    Vi_out = viF.reshape(N, _NM, _NCH).transpose(0, 2, 1)
    return xiF, Vi_out


# speedup vs baseline: 2.8960x; 1.5422x over previous
import math
from functools import partial

import jax
import jax.numpy as jnp
import numpy as np
from jax.experimental import pallas as pl
from jax.experimental.pallas import tpu as pltpu
from jax.experimental.pallas import tpu_sc as plsc

_LMAX = 2
_NCH = 16
_MSG = 8
_RDIM = 8
_DIM = 128
_CUTOFF = 5.0
_NCD = _MSG * _RDIM
_NM = (_LMAX + 1) ** 2
_L_OF_M = [l for l in range(_LMAX + 1) for _ in range(2 * l + 1)]

_WIN = 128
_G0W = 128
_G1W = 256
_EB = 1280
_NB = 1000
_GW = 128

_f32 = jnp.float32


def _cg_coef(l1, m1, l2, m2, l3, m3):
    if m1 + m2 != m3 or l3 < abs(l1 - l2) or l3 > l1 + l2:
        return 0.0
    f = math.factorial
    pref = ((2 * l3 + 1) * f(l3 + l1 - l2) * f(l3 - l1 + l2) * f(l1 + l2 - l3)
            / f(l1 + l2 + l3 + 1)) ** 0.5
    pref *= (f(l3 + m3) * f(l3 - m3) * f(l1 - m1) * f(l1 + m1) * f(l2 - m2)
             * f(l2 + m2)) ** 0.5
    s = 0.0
    for k in range(0, l1 + l2 - l3 + 1):
        d = [k, l1 + l2 - l3 - k, l1 - m1 - k, l2 + m2 - k, l3 - l2 + m1 + k,
             l3 - l1 - m2 + k]
        if min(d) < 0:
            continue
        den = 1.0
        for q in d:
            den *= f(q)
        s += (-1) ** k / den
    return pref * s


def _u_mat(l):
    U = np.zeros((2 * l + 1, 2 * l + 1), dtype=complex)
    for m in range(-l, l + 1):
        i = m + l
        if m == 0:
            U[i, l] = 1.0
        elif m > 0:
            U[i, l + m] = (-1) ** m / np.sqrt(2.0)
            U[i, l - m] = 1.0 / np.sqrt(2.0)
        else:
            mm = -m
            U[i, l - mm] = 1j / np.sqrt(2.0)
            U[i, l + mm] = -1j * ((-1) ** mm) / np.sqrt(2.0)
    return U


def _real_cg_block(l1, l2, l3):
    U1, U2, U3 = _u_mat(l1), _u_mat(l2), _u_mat(l3)
    C = np.zeros((2 * l1 + 1, 2 * l2 + 1, 2 * l3 + 1), dtype=complex)
    for a in range(2 * l1 + 1):
        for b in range(2 * l2 + 1):
            for c in range(2 * l3 + 1):
                s = 0.0 + 0.0j
                for m1 in range(-l1, l1 + 1):
                    for m2 in range(-l2, l2 + 1):
                        m3 = m1 + m2
                        if abs(m3) > l3:
                            continue
                        s += (np.conj(U1[a, m1 + l1]) * np.conj(U2[b, m2 + l2])
                              * U3[c, m3 + l3] * _cg_coef(l1, m1, l2, m2, l3, m3))
                C[a, b, c] = s
    return (C.real + C.imag).astype(np.float32)


def _build_cg_groups():
    paths = [(l1, l2, l3)
             for l1 in range(_LMAX + 1) for l2 in range(_LMAX + 1)
             for l3 in range(abs(l1 - l2), min(_LMAX, l1 + l2) + 1)]
    groups = {}
    for p, (l1, l2, l3) in enumerate(paths):
        blk = _real_cg_block(l1, l2, l3)
        for a in range(2 * l1 + 1):
            for b in range(2 * l2 + 1):
                for c in range(2 * l3 + 1):
                    v = float(blk[a, b, c])
                    if abs(v) < 1e-10:
                        continue
                    i, j, k = l1 * l1 + a, l2 * l2 + b, l3 * l3 + c
                    groups.setdefault((i, j), []).append((k, p, v))
    return sorted(groups.items()), len(paths)


_CG_GROUPS, _NPATHS = _build_cg_groups()
_NGRP = len(_CG_GROUPS)
_GX = _NGRP * _NCH


def _build_tp_tables():
    EV = np.zeros((_NM * _NCH, _GX), np.float32)
    EH = np.zeros((_NM * _NCH, _GX), np.float32)
    B = np.zeros((_NPATHS, _GX, _NM * _NCH), np.float32)
    for g, ((i, j), terms) in enumerate(_CG_GROUPS):
        for c in range(_NCH):
            EV[i * _NCH + c, g * _NCH + c] = 1.0
            EH[j * _NCH + c, g * _NCH + c] = 1.0
        for (k, p, v) in terms:
            for c in range(_NCH):
                B[p, g * _NCH + c, k * _NCH + c] += v
    return EV, EH, B


_EV_NP, _EH_NP, _B_NP = _build_tp_tables()
_LANE16 = np.tile(np.arange(_NCH), _NM)


def _rep_mats():
    repM = np.zeros((_MSG, _NCD * _NM), np.float32)
    repR = np.zeros((_RDIM, _NCD * _NM), np.float32)
    repY = np.zeros((_NM, _NCD * _NM), np.float32)
    for m in range(_NM):
        for c in range(_MSG):
            for r in range(_RDIM):
                col = m * _NCD + c * _RDIM + r
                repM[c, col] = 1.0
                repR[r, col] = 1.0
                repY[m, col] = 1.0
    return repM, repR, repY


_REPM_NP, _REPR_NP, _REPY_NP = _rep_mats()


def _geom_kernel(d_ref, sw_ref, x_ref, y_ref, z_ref, rb_ref, ysh_ref):
    d = d_ref[...]
    inv = 1.0 / d
    scale = np.float32(math.sqrt(2.0 / _CUTOFF)) * inv * sw_ref[...]
    t = d * np.float32(math.pi / _CUTOFF)
    s1 = jnp.sin(t)
    c2 = 2.0 * jnp.cos(t)
    sns = [s1, c2 * s1]
    for _ in range(_RDIM - 2):
        sns.append(c2 * sns[-1] - sns[-2])
    rb_ref[...] = jnp.concatenate([s * scale for s in sns], axis=1)
    x = x_ref[...] * inv
    y = y_ref[...] * inv
    z = z_ref[...] * inv
    c1 = np.float32(math.sqrt(3.0))
    cA = np.float32(math.sqrt(15.0))
    cB = np.float32(math.sqrt(5.0) / 2.0)
    ysh_ref[...] = jnp.concatenate([
        jnp.ones_like(x), c1 * y, c1 * z, c1 * x,
        cA * x * y, cA * y * z, cB * (3.0 * z * z - 1.0), cA * x * z,
        np.float32(0.5) * cA * (x * x - y * y)], axis=1)


def _mm(a, b):
    return jnp.dot(a, b, preferred_element_type=_f32)


def _accumulate_sorted(dens_ref, src, contrib, nb):
    iot = jax.lax.broadcasted_iota(jnp.int32, (nb, 1), 0)

    def cond(s):
        return s < nb

    def body(s):
        masked = jnp.where(iot >= s, src, jnp.int32(2 ** 30))
        w0 = (jnp.min(masked) // 8) * 8
        rel = src - w0
        ok = (iot >= s) & (rel < _WIN)
        S = ((rel == jax.lax.broadcasted_iota(jnp.int32, (nb, _WIN), 1))
             & ok).astype(_f32)
        upd = jax.lax.dot_general(S, contrib, (((0,), (0,)), ((), ())),
                                  preferred_element_type=_f32)
        dens_ref[pl.ds(w0, _WIN), :] = dens_ref[pl.ds(w0, _WIN), :] + upd
        return s + jnp.sum(ok.astype(jnp.int32))

    jax.lax.while_loop(cond, body, jnp.int32(0))


def _seg0_kernel(src_ref, rb_ref, ysh_ref, g_ref, repM_ref, repR_ref,
                 repY_ref, dens_ref):
    @pl.when(pl.program_id(0) == 0)
    def _():
        dens_ref[...] = jnp.zeros(dens_ref.shape, _f32)

    nb = src_ref.shape[0]
    mi = g_ref[...][:, 0:_MSG]
    contrib = (_mm(ysh_ref[...], repY_ref[...]) * _mm(mi, repM_ref[...])
               * _mm(rb_ref[...], repR_ref[...]))
    _accumulate_sorted(dens_ref, src_ref[...], contrib, nb)


def _seg1_kernel(src_ref, rb_ref, g_ref, wrho_bd_ref, repM_ref, repR_ref,
                 dens_ref):
    @pl.when(pl.program_id(0) == 0)
    def _():
        dens_ref[...] = jnp.zeros(dens_ref.shape, _f32)

    nb = src_ref.shape[0]
    g = g_ref[...]
    mi = g[:, 0:_MSG]
    Rfull = _mm(g[:, _MSG:_MSG + _NCH * _NM], wrho_bd_ref[...])
    contrib = _mm(mi, repM_ref[...]) * _mm(rb_ref[...], repR_ref[...]) * Rfull
    _accumulate_sorted(dens_ref, src_ref[...], contrib, nb)


def _node_pre_kernel(spec_ref, wspec_ref, bspec_ref, wmsg_ref, xi_ref, mi_ref):
    sp = spec_ref[...]
    nb = sp.shape[0]
    enc = (sp == jax.lax.broadcasted_iota(jnp.int32, (nb, 64), 1)).astype(_f32)
    xi = jnp.dot(enc, wspec_ref[...], preferred_element_type=_f32) + bspec_ref[...]
    xi_ref[...] = xi
    mi = jnp.dot(xi, wmsg_ref[...], preferred_element_type=_f32)
    mi_ref[...] = jnp.concatenate(
        [mi, jnp.zeros((nb, _G0W - _MSG), _f32)], axis=1)


def _node_layer(dens, xi, vi, wdma_bd, wdmb_bd, EV, EH, Ma, Mb,
                wla, bla, wlb, blb):
    Hia = _mm(dens, wdma_bd)
    Lia = _mm(_mm(vi, EV) * _mm(Hia, EH), Ma)
    vi = vi + Lia
    Hib = _mm(dens, wdmb_bd)
    Lib = _mm(_mm(vi, EV) * _mm(Hib, EH), Mb)
    vi = vi + Lib
    h = jnp.concatenate(
        [xi, dens[:, 0:_NCD], Lia[:, 0:_NCH], Lib[:, 0:_NCH]], axis=1)
    pre = _mm(h, wla) + bla
    act = pre * jax.nn.sigmoid(pre)
    return xi + _mm(act, wlb) + blb, vi


def _node_mid_kernel(dens_ref, xi_ref, wvibd_ref, wdma_ref, wdmb_ref,
                     ev_ref, eh_ref, ma_ref, mb_ref, wla_ref, bla_ref,
                     wlb_ref, blb_ref, wmsg_ref, xi1_ref, vi_ref, t1_ref):
    dens = dens_ref[...]
    vi0 = _mm(dens, wvibd_ref[...])
    xi1, vi = _node_layer(
        dens, xi_ref[...], vi0, wdma_ref[...], wdmb_ref[...], ev_ref[...],
        eh_ref[...], ma_ref[...], mb_ref[...], wla_ref[...], bla_ref[...],
        wlb_ref[...], blb_ref[...])
    xi1_ref[...] = xi1
    vi_ref[...] = vi
    mi1 = _mm(xi1, wmsg_ref[...])
    pad = _G1W - _MSG - _NCH * _NM
    t1_ref[...] = jnp.concatenate(
        [mi1, vi, jnp.zeros((mi1.shape[0], pad), _f32)], axis=1)


def _node_fin_kernel(d0_ref, d1_ref, xi_ref, vi_ref, wdma_ref, wdmb_ref,
                     ev_ref, eh_ref, ma_ref, mb_ref, wla_ref, bla_ref,
                     wlb_ref, blb_ref, xiF_ref, viF_ref):
    dens = d0_ref[...] + d1_ref[...]
    xiF, vi = _node_layer(
        dens, xi_ref[...], vi_ref[...], wdma_ref[...], wdmb_ref[...],
        ev_ref[...], eh_ref[...], ma_ref[...], mb_ref[...], wla_ref[...],
        bla_ref[...], wlb_ref[...], blb_ref[...])
    xiF_ref[...] = xiF
    viF_ref[...] = vi


def _sc_gather(table, idx):
    n_idx = idx.shape[0]
    width = table.shape[1]
    idx2 = idx.reshape(1, n_idx)
    mesh = plsc.VectorSubcoreMesh(core_axis_name="c", subcore_axis_name="s")

    @partial(pl.kernel,
             out_type=jax.ShapeDtypeStruct((n_idx, width), table.dtype),
             mesh=mesh)
    def gk(tab_hbm, i_hbm, o_hbm):
        def body(i_vmem, o_vmem):
            pltpu.sync_copy(tab_hbm.at[i_vmem.at[0]], o_vmem)

        pltpu.emit_pipeline(
            body,
            grid=(n_idx // _GW,),
            in_specs=[pl.BlockSpec((1, _GW), lambda i: (0, i))],
            out_specs=[pl.BlockSpec((_GW, width), lambda i: (i, 0))],
            core_axis_name=("c", "s"),
            dimension_semantics=(pltpu.PARALLEL,),
        )(i_hbm, o_hbm)

    return gk(table, idx2)


def _tc_params(vmem_mb, parallel=False):
    sem = ("parallel",) if parallel else ("arbitrary",)
    return pltpu.CompilerParams(dimension_semantics=sem,
                                vmem_limit_bytes=vmem_mb * 1024 * 1024)


def kernel(species, edge_src, edge_dst, distances, vec, switch,
           W_spec, b_spec, W_msg0, b_msg0, W_msg1, b_msg1, W_vi, W_rho,
           W_dm00, W_dm01, W_dm10, W_dm11, W_tp00, W_tp01, W_tp10, W_tp11,
           W_lat0a, b_lat0a, W_lat0b, b_lat0b, W_lat1a, b_lat1a, W_lat1b,
           b_lat1b):
    N = species.shape[0]
    E = edge_src.shape[0]
    n_pad = ((N + _WIN + 7) // 8) * 8
    e_pad = ((E + _EB - 1) // _EB) * _EB
    nb = _NB if N % _NB == 0 else N
    n_grid = N // nb

    src2 = edge_src.astype(jnp.int32).reshape(E, 1)
    dst1 = edge_dst.astype(jnp.int32)
    d2 = distances.astype(_f32).reshape(E, 1)
    vec2 = vec.astype(_f32)
    sw2 = switch.astype(_f32).reshape(E, 1)
    if e_pad != E:
        p = e_pad - E
        src2 = jnp.concatenate([src2, jnp.full((p, 1), N, jnp.int32)])
        dst1 = jnp.concatenate([dst1, jnp.zeros((p,), jnp.int32)])
        d2 = jnp.concatenate([d2, jnp.ones((p, 1), _f32)])
        vec2 = jnp.concatenate([vec2, jnp.ones((p, 3), _f32)])
        sw2 = jnp.concatenate([sw2, jnp.zeros((p, 1), _f32)])

    lom = np.asarray(_L_OF_M)
    wspec_p = jnp.concatenate(
        [W_spec, jnp.zeros((64 - W_spec.shape[0], _DIM), _f32)], axis=0)
    bspec2 = b_spec.reshape(1, _DIM)
    wviT = jnp.transpose(W_vi[lom], (0, 2, 1))
    wrhoT = jnp.transpose(W_rho[lom], (0, 2, 1))
    bd = jax.scipy.linalg.block_diag
    wvi_bd = bd(*[wviT[m] for m in range(_NM)])
    wrho_bd = bd(*[wrhoT[m] for m in range(_NM)])
    wdm00_bd = bd(*([W_dm00.T] * _NM))
    wdm01_bd = bd(*([W_dm01.T] * _NM))
    wdm10_bd = bd(*([W_dm10.T] * _NM))
    wdm11_bd = bd(*([W_dm11.T] * _NM))
    Bt = jnp.asarray(_B_NP)
    Ma0 = jnp.einsum('pgk,pk->gk', Bt, W_tp00[:, _LANE16])
    Mb0 = jnp.einsum('pgk,pk->gk', Bt, W_tp01[:, _LANE16])
    Ma1 = jnp.einsum('pgk,pk->gk', Bt, W_tp10[:, _LANE16])
    Mb1 = jnp.einsum('pgk,pk->gk', Bt, W_tp11[:, _LANE16])
    EV, EH = jnp.asarray(_EV_NP), jnp.asarray(_EH_NP)
    repM, repR, repY = (jnp.asarray(_REPM_NP), jnp.asarray(_REPR_NP),
                        jnp.asarray(_REPY_NP))
    bl0a, bl0b = b_lat0a.reshape(1, -1), b_lat0b.reshape(1, -1)
    bl1a, bl1b = b_lat1a.reshape(1, -1), b_lat1b.reshape(1, -1)

    ebs = lambda w: pl.BlockSpec((_EB, w), lambda i: (i, 0))
    nbs = lambda w: pl.BlockSpec((nb, w), lambda i: (i, 0))
    full = lambda *s: pl.BlockSpec(s, lambda i: tuple(0 for _ in s))
    dspec = pl.BlockSpec((n_pad, _NCD * _NM), lambda i: (0, 0))

    rows = e_pad // 128
    rb_pk, ysh_pk = pl.pallas_call(
        _geom_kernel,
        grid=(1,),
        in_specs=[full(rows, 128)] * 5,
        out_specs=[full(rows, _RDIM * 128), full(rows, _NM * 128)],
        out_shape=[jax.ShapeDtypeStruct((rows, _RDIM * 128), _f32),
                   jax.ShapeDtypeStruct((rows, _NM * 128), _f32)],
        compiler_params=_tc_params(40, parallel=True),
    )(d2.reshape(rows, 128), sw2.reshape(rows, 128),
      vec2[:, 0].reshape(rows, 128), vec2[:, 1].reshape(rows, 128),
      vec2[:, 2].reshape(rows, 128))
    rb_e = (rb_pk.reshape(rows, _RDIM, 128).transpose(0, 2, 1)
            .reshape(e_pad, _RDIM))
    ysh_e = (ysh_pk.reshape(rows, _NM, 128).transpose(0, 2, 1)
             .reshape(e_pad, _NM))

    xi0, mi0p = pl.pallas_call(
        _node_pre_kernel,
        grid=(n_grid,),
        in_specs=[nbs(1), full(64, _DIM), full(1, _DIM), full(_DIM, _MSG)],
        out_specs=[nbs(_DIM), nbs(_G0W)],
        out_shape=[jax.ShapeDtypeStruct((N, _DIM), _f32),
                   jax.ShapeDtypeStruct((N, _G0W), _f32)],
        compiler_params=_tc_params(64, parallel=True),
    )(species.astype(jnp.int32).reshape(N, 1), wspec_p, bspec2, W_msg0)

    g0 = _sc_gather(mi0p, dst1)

    dens0 = pl.pallas_call(
        _seg0_kernel,
        grid=(e_pad // _EB,),
        in_specs=[ebs(1), ebs(_RDIM), ebs(_NM), ebs(_G0W),
                  full(_MSG, _NCD * _NM), full(_RDIM, _NCD * _NM),
                  full(_NM, _NCD * _NM)],
        out_specs=dspec,
        out_shape=jax.ShapeDtypeStruct((n_pad, _NCD * _NM), _f32),
        compiler_params=_tc_params(56),
    )(src2, rb_e, ysh_e, g0, repM, repR, repY)

    xi1, vi1, t1 = pl.pallas_call(
        _node_mid_kernel,
        grid=(n_grid,),
        in_specs=[nbs(_NCD * _NM), nbs(_DIM),
                  full(_NCD * _NM, _NCH * _NM), full(_NCD * _NM, _NCH * _NM),
                  full(_NCD * _NM, _NCH * _NM),
                  full(_NCH * _NM, _GX), full(_NCH * _NM, _GX),
                  full(_GX, _NCH * _NM), full(_GX, _NCH * _NM),
                  full(_DIM + _NCD + 2 * _NCH, _DIM), full(1, _DIM),
                  full(_DIM, _DIM), full(1, _DIM), full(_DIM, _MSG)],
        out_specs=[nbs(_DIM), nbs(_NCH * _NM), nbs(_G1W)],
        out_shape=[jax.ShapeDtypeStruct((N, _DIM), _f32),
                   jax.ShapeDtypeStruct((N, _NCH * _NM), _f32),
                   jax.ShapeDtypeStruct((N, _G1W), _f32)],
        compiler_params=_tc_params(64, parallel=True),
    )(dens0, xi0, wvi_bd, wdm00_bd, wdm01_bd, EV, EH, Ma0, Mb0,
      W_lat0a, bl0a, W_lat0b, bl0b, W_msg1)

    g1 = _sc_gather(t1, dst1)

    dens1 = pl.pallas_call(
        _seg1_kernel,
        grid=(e_pad // _EB,),
        in_specs=[ebs(1), ebs(_RDIM), ebs(_G1W),
                  full(_NCH * _NM, _NCD * _NM), full(_MSG, _NCD * _NM),
                  full(_RDIM, _NCD * _NM)],
        out_specs=dspec,
        out_shape=jax.ShapeDtypeStruct((n_pad, _NCD * _NM), _f32),
        compiler_params=_tc_params(56),
    )(src2, rb_e, g1, wrho_bd, repM, repR)

    xiF, viF = pl.pallas_call(
        _node_fin_kernel,
        grid=(n_grid,),
        in_specs=[nbs(_NCD * _NM), nbs(_NCD * _NM), nbs(_DIM),
                  nbs(_NCH * _NM),
                  full(_NCD * _NM, _NCH * _NM), full(_NCD * _NM, _NCH * _NM),
                  full(_NCH * _NM, _GX), full(_NCH * _NM, _GX),
                  full(_GX, _NCH * _NM), full(_GX, _NCH * _NM),
                  full(_DIM + _NCD + 2 * _NCH, _DIM), full(1, _DIM),
                  full(_DIM, _DIM), full(1, _DIM)],
        out_specs=[nbs(_DIM), nbs(_NCH * _NM)],
        out_shape=[jax.ShapeDtypeStruct((N, _DIM), _f32),
                   jax.ShapeDtypeStruct((N, _NCH * _NM), _f32)],
        compiler_params=_tc_params(64, parallel=True),
    )(dens0, dens1, xi1, vi1, wdm10_bd, wdm11_bd, EV, EH, Ma1, Mb1,
      W_lat1a, bl1a, W_lat1b, bl1b)

    Vi_out = viF.reshape(N, _NM, _NCH).transpose(0, 2, 1)
    return xiF, Vi_out


# speedup vs baseline: 3.1865x; 1.1003x over previous
import math
from functools import partial

import jax
import jax.numpy as jnp
import numpy as np
from jax.experimental import pallas as pl
from jax.experimental.pallas import tpu as pltpu
from jax.experimental.pallas import tpu_sc as plsc

_LMAX = 2
_NCH = 16
_MSG = 8
_RDIM = 8
_DIM = 128
_CUTOFF = 5.0
_NCD = _MSG * _RDIM
_NM = (_LMAX + 1) ** 2
_L_OF_M = [l for l in range(_LMAX + 1) for _ in range(2 * l + 1)]

_WIN = 128
_G0W = 128
_G1W = 256
_EB = 1280
_NB = 1000
_GW = 128

_f32 = jnp.float32


def _cg_coef(l1, m1, l2, m2, l3, m3):
    if m1 + m2 != m3 or l3 < abs(l1 - l2) or l3 > l1 + l2:
        return 0.0
    f = math.factorial
    pref = ((2 * l3 + 1) * f(l3 + l1 - l2) * f(l3 - l1 + l2) * f(l1 + l2 - l3)
            / f(l1 + l2 + l3 + 1)) ** 0.5
    pref *= (f(l3 + m3) * f(l3 - m3) * f(l1 - m1) * f(l1 + m1) * f(l2 - m2)
             * f(l2 + m2)) ** 0.5
    s = 0.0
    for k in range(0, l1 + l2 - l3 + 1):
        d = [k, l1 + l2 - l3 - k, l1 - m1 - k, l2 + m2 - k, l3 - l2 + m1 + k,
             l3 - l1 - m2 + k]
        if min(d) < 0:
            continue
        den = 1.0
        for q in d:
            den *= f(q)
        s += (-1) ** k / den
    return pref * s


def _u_mat(l):
    U = np.zeros((2 * l + 1, 2 * l + 1), dtype=complex)
    for m in range(-l, l + 1):
        i = m + l
        if m == 0:
            U[i, l] = 1.0
        elif m > 0:
            U[i, l + m] = (-1) ** m / np.sqrt(2.0)
            U[i, l - m] = 1.0 / np.sqrt(2.0)
        else:
            mm = -m
            U[i, l - mm] = 1j / np.sqrt(2.0)
            U[i, l + mm] = -1j * ((-1) ** mm) / np.sqrt(2.0)
    return U


def _real_cg_block(l1, l2, l3):
    U1, U2, U3 = _u_mat(l1), _u_mat(l2), _u_mat(l3)
    C = np.zeros((2 * l1 + 1, 2 * l2 + 1, 2 * l3 + 1), dtype=complex)
    for a in range(2 * l1 + 1):
        for b in range(2 * l2 + 1):
            for c in range(2 * l3 + 1):
                s = 0.0 + 0.0j
                for m1 in range(-l1, l1 + 1):
                    for m2 in range(-l2, l2 + 1):
                        m3 = m1 + m2
                        if abs(m3) > l3:
                            continue
                        s += (np.conj(U1[a, m1 + l1]) * np.conj(U2[b, m2 + l2])
                              * U3[c, m3 + l3] * _cg_coef(l1, m1, l2, m2, l3, m3))
                C[a, b, c] = s
    return (C.real + C.imag).astype(np.float32)


def _build_cg_groups():
    paths = [(l1, l2, l3)
             for l1 in range(_LMAX + 1) for l2 in range(_LMAX + 1)
             for l3 in range(abs(l1 - l2), min(_LMAX, l1 + l2) + 1)]
    groups = {}
    for p, (l1, l2, l3) in enumerate(paths):
        blk = _real_cg_block(l1, l2, l3)
        for a in range(2 * l1 + 1):
            for b in range(2 * l2 + 1):
                for c in range(2 * l3 + 1):
                    v = float(blk[a, b, c])
                    if abs(v) < 1e-10:
                        continue
                    i, j, k = l1 * l1 + a, l2 * l2 + b, l3 * l3 + c
                    groups.setdefault((i, j), []).append((k, p, v))
    return sorted(groups.items()), len(paths)


_CG_GROUPS, _NPATHS = _build_cg_groups()
_NGRP = len(_CG_GROUPS)
_GX = _NGRP * _NCH


def _build_tp_tables():
    EV = np.zeros((_NM * _NCH, _GX), np.float32)
    EH = np.zeros((_NM * _NCH, _GX), np.float32)
    B = np.zeros((_NPATHS, _GX, _NM * _NCH), np.float32)
    for g, ((i, j), terms) in enumerate(_CG_GROUPS):
        for c in range(_NCH):
            EV[i * _NCH + c, g * _NCH + c] = 1.0
            EH[j * _NCH + c, g * _NCH + c] = 1.0
        for (k, p, v) in terms:
            for c in range(_NCH):
                B[p, g * _NCH + c, k * _NCH + c] += v
    return EV, EH, B


_EV_NP, _EH_NP, _B_NP = _build_tp_tables()
_LANE16 = np.tile(np.arange(_NCH), _NM)


def _rep_mats():
    repM = np.zeros((_MSG, _NCD * _NM), np.float32)
    repR = np.zeros((_RDIM, _NCD * _NM), np.float32)
    repY = np.zeros((_NM, _NCD * _NM), np.float32)
    for m in range(_NM):
        for c in range(_MSG):
            for r in range(_RDIM):
                col = m * _NCD + c * _RDIM + r
                repM[c, col] = 1.0
                repR[r, col] = 1.0
                repY[m, col] = 1.0
    return repM, repR, repY


_REPM_NP, _REPR_NP, _REPY_NP = _rep_mats()


def _geom_kernel(d_ref, sw_ref, x_ref, y_ref, z_ref, rb_ref, ysh_ref):
    d = d_ref[...]
    inv = 1.0 / d
    scale = np.float32(math.sqrt(2.0 / _CUTOFF)) * inv * sw_ref[...]
    t = d * np.float32(math.pi / _CUTOFF)
    s1 = jnp.sin(t)
    c2 = 2.0 * jnp.cos(t)
    sns = [s1, c2 * s1]
    for _ in range(_RDIM - 2):
        sns.append(c2 * sns[-1] - sns[-2])
    rb_ref[...] = jnp.concatenate([s * scale for s in sns], axis=1)
    x = x_ref[...] * inv
    y = y_ref[...] * inv
    z = z_ref[...] * inv
    c1 = np.float32(math.sqrt(3.0))
    cA = np.float32(math.sqrt(15.0))
    cB = np.float32(math.sqrt(5.0) / 2.0)
    ysh_ref[...] = jnp.concatenate([
        jnp.ones_like(x), c1 * y, c1 * z, c1 * x,
        cA * x * y, cA * y * z, cB * (3.0 * z * z - 1.0), cA * x * z,
        np.float32(0.5) * cA * (x * x - y * y)], axis=1)


def _mm(a, b):
    return jnp.dot(a, b, preferred_element_type=_f32)


def _accumulate_sorted(dens_ref, src_ref, contrib, nb):
    src = src_ref[...]
    lane = jax.lax.broadcasted_iota(jnp.int32, (nb, _WIN), 1)

    def cond(s):
        return s < nb

    def body(s):
        first = src_ref[pl.ds(s, 1), :][0, 0]
        w0 = (first // 8) * 8
        off = first - w0
        rel = src - w0
        S = ((rel == lane) & (lane >= off)).astype(_f32)
        upd = jax.lax.dot_general(S, contrib, (((0,), (0,)), ((), ())),
                                  preferred_element_type=_f32)
        dens_ref[pl.ds(w0, _WIN), :] = dens_ref[pl.ds(w0, _WIN), :] + upd
        return jnp.sum((src < w0 + _WIN).astype(jnp.int32))

    jax.lax.while_loop(cond, body, jnp.int32(0))


def _seg0_kernel(src_ref, rb_ref, ysh_ref, g_ref, repM_ref, repR_ref,
                 repY_ref, dens_ref):
    @pl.when(pl.program_id(0) == 0)
    def _():
        dens_ref[...] = jnp.zeros(dens_ref.shape, _f32)

    nb = src_ref.shape[0]
    mi = g_ref[...][:, 0:_MSG]
    contrib = (_mm(ysh_ref[...], repY_ref[...]) * _mm(mi, repM_ref[...])
               * _mm(rb_ref[...], repR_ref[...]))
    _accumulate_sorted(dens_ref, src_ref, contrib, nb)


def _seg1_kernel(src_ref, rb_ref, g_ref, wrho_bd_ref, repM_ref, repR_ref,
                 dens_ref):
    @pl.when(pl.program_id(0) == 0)
    def _():
        dens_ref[...] = jnp.zeros(dens_ref.shape, _f32)

    nb = src_ref.shape[0]
    g = g_ref[...]
    mi = g[:, 0:_MSG]
    Rfull = _mm(g[:, _MSG:_MSG + _NCH * _NM], wrho_bd_ref[...])
    contrib = _mm(mi, repM_ref[...]) * _mm(rb_ref[...], repR_ref[...]) * Rfull
    _accumulate_sorted(dens_ref, src_ref, contrib, nb)


def _node_pre_kernel(spec_ref, wspec_ref, bspec_ref, wmsg_ref, xi_ref, mi_ref):
    sp = spec_ref[...]
    nb = sp.shape[0]
    enc = (sp == jax.lax.broadcasted_iota(jnp.int32, (nb, 64), 1)).astype(_f32)
    xi = jnp.dot(enc, wspec_ref[...], preferred_element_type=_f32) + bspec_ref[...]
    xi_ref[...] = xi
    mi = jnp.dot(xi, wmsg_ref[...], preferred_element_type=_f32)
    mi_ref[...] = jnp.concatenate(
        [mi, jnp.zeros((nb, _G0W - _MSG), _f32)], axis=1)


def _node_layer(dens, xi, vi, wdma_bd, wdmb_bd, EV, EH, Ma, Mb,
                wla, bla, wlb, blb):
    Hia = _mm(dens, wdma_bd)
    Lia = _mm(_mm(vi, EV) * _mm(Hia, EH), Ma)
    vi = vi + Lia
    Hib = _mm(dens, wdmb_bd)
    Lib = _mm(_mm(vi, EV) * _mm(Hib, EH), Mb)
    vi = vi + Lib
    h = jnp.concatenate(
        [xi, dens[:, 0:_NCD], Lia[:, 0:_NCH], Lib[:, 0:_NCH]], axis=1)
    pre = _mm(h, wla) + bla
    act = pre * jax.nn.sigmoid(pre)
    return xi + _mm(act, wlb) + blb, vi


def _node_mid_kernel(dens_ref, xi_ref, wvibd_ref, wdma_ref, wdmb_ref,
                     ev_ref, eh_ref, ma_ref, mb_ref, wla_ref, bla_ref,
                     wlb_ref, blb_ref, wmsg_ref, xi1_ref, vi_ref, t1_ref):
    dens = dens_ref[...]
    vi0 = _mm(dens, wvibd_ref[...])
    xi1, vi = _node_layer(
        dens, xi_ref[...], vi0, wdma_ref[...], wdmb_ref[...], ev_ref[...],
        eh_ref[...], ma_ref[...], mb_ref[...], wla_ref[...], bla_ref[...],
        wlb_ref[...], blb_ref[...])
    xi1_ref[...] = xi1
    vi_ref[...] = vi
    mi1 = _mm(xi1, wmsg_ref[...])
    pad = _G1W - _MSG - _NCH * _NM
    t1_ref[...] = jnp.concatenate(
        [mi1, vi, jnp.zeros((mi1.shape[0], pad), _f32)], axis=1)


def _node_fin_kernel(d0_ref, d1_ref, xi_ref, vi_ref, wdma_ref, wdmb_ref,
                     ev_ref, eh_ref, ma_ref, mb_ref, wla_ref, bla_ref,
                     wlb_ref, blb_ref, xiF_ref, viF_ref):
    dens = d0_ref[...] + d1_ref[...]
    xiF, vi = _node_layer(
        dens, xi_ref[...], vi_ref[...], wdma_ref[...], wdmb_ref[...],
        ev_ref[...], eh_ref[...], ma_ref[...], mb_ref[...], wla_ref[...],
        bla_ref[...], wlb_ref[...], blb_ref[...])
    xiF_ref[...] = xiF
    viF_ref[...] = vi


def _sc_gather(table, idx):
    n_idx = idx.shape[0]
    width = table.shape[1]
    idx2 = idx.reshape(1, n_idx)
    mesh = plsc.VectorSubcoreMesh(core_axis_name="c", subcore_axis_name="s")

    @partial(pl.kernel,
             out_type=jax.ShapeDtypeStruct((n_idx, width), table.dtype),
             mesh=mesh)
    def gk(tab_hbm, i_hbm, o_hbm):
        def body(i_vmem, o_vmem):
            pltpu.sync_copy(tab_hbm.at[i_vmem.at[0]], o_vmem)

        pltpu.emit_pipeline(
            body,
            grid=(n_idx // _GW,),
            in_specs=[pl.BlockSpec((1, _GW), lambda i: (0, i))],
            out_specs=[pl.BlockSpec((_GW, width), lambda i: (i, 0))],
            core_axis_name=("c", "s"),
            dimension_semantics=(pltpu.PARALLEL,),
        )(i_hbm, o_hbm)

    return gk(table, idx2)


def _tc_params(vmem_mb, parallel=False):
    sem = ("parallel",) if parallel else ("arbitrary",)
    return pltpu.CompilerParams(dimension_semantics=sem,
                                vmem_limit_bytes=vmem_mb * 1024 * 1024)


def kernel(species, edge_src, edge_dst, distances, vec, switch,
           W_spec, b_spec, W_msg0, b_msg0, W_msg1, b_msg1, W_vi, W_rho,
           W_dm00, W_dm01, W_dm10, W_dm11, W_tp00, W_tp01, W_tp10, W_tp11,
           W_lat0a, b_lat0a, W_lat0b, b_lat0b, W_lat1a, b_lat1a, W_lat1b,
           b_lat1b):
    N = species.shape[0]
    E = edge_src.shape[0]
    n_pad = ((N + _WIN + 7) // 8) * 8
    e_pad = ((E + _EB - 1) // _EB) * _EB
    nb = _NB if N % _NB == 0 else N
    n_grid = N // nb

    src2 = edge_src.astype(jnp.int32).reshape(E, 1)
    dst1 = edge_dst.astype(jnp.int32)
    d2 = distances.astype(_f32).reshape(E, 1)
    vec2 = vec.astype(_f32)
    sw2 = switch.astype(_f32).reshape(E, 1)
    if e_pad != E:
        p = e_pad - E
        src2 = jnp.concatenate([src2, jnp.full((p, 1), N, jnp.int32)])
        dst1 = jnp.concatenate([dst1, jnp.zeros((p,), jnp.int32)])
        d2 = jnp.concatenate([d2, jnp.ones((p, 1), _f32)])
        vec2 = jnp.concatenate([vec2, jnp.ones((p, 3), _f32)])
        sw2 = jnp.concatenate([sw2, jnp.zeros((p, 1), _f32)])

    lom = np.asarray(_L_OF_M)
    wspec_p = jnp.concatenate(
        [W_spec, jnp.zeros((64 - W_spec.shape[0], _DIM), _f32)], axis=0)
    bspec2 = b_spec.reshape(1, _DIM)
    wviT = jnp.transpose(W_vi[lom], (0, 2, 1))
    wrhoT = jnp.transpose(W_rho[lom], (0, 2, 1))
    bd = jax.scipy.linalg.block_diag
    wvi_bd = bd(*[wviT[m] for m in range(_NM)])
    wrho_bd = bd(*[wrhoT[m] for m in range(_NM)])
    wdm00_bd = bd(*([W_dm00.T] * _NM))
    wdm01_bd = bd(*([W_dm01.T] * _NM))
    wdm10_bd = bd(*([W_dm10.T] * _NM))
    wdm11_bd = bd(*([W_dm11.T] * _NM))
    Bt = jnp.asarray(_B_NP)
    Ma0 = jnp.einsum('pgk,pk->gk', Bt, W_tp00[:, _LANE16])
    Mb0 = jnp.einsum('pgk,pk->gk', Bt, W_tp01[:, _LANE16])
    Ma1 = jnp.einsum('pgk,pk->gk', Bt, W_tp10[:, _LANE16])
    Mb1 = jnp.einsum('pgk,pk->gk', Bt, W_tp11[:, _LANE16])
    EV, EH = jnp.asarray(_EV_NP), jnp.asarray(_EH_NP)
    repM, repR, repY = (jnp.asarray(_REPM_NP), jnp.asarray(_REPR_NP),
                        jnp.asarray(_REPY_NP))
    bl0a, bl0b = b_lat0a.reshape(1, -1), b_lat0b.reshape(1, -1)
    bl1a, bl1b = b_lat1a.reshape(1, -1), b_lat1b.reshape(1, -1)

    ebs = lambda w: pl.BlockSpec((_EB, w), lambda i: (i, 0))
    nbs = lambda w: pl.BlockSpec((nb, w), lambda i: (i, 0))
    full = lambda *s: pl.BlockSpec(s, lambda i: tuple(0 for _ in s))
    dspec = pl.BlockSpec((n_pad, _NCD * _NM), lambda i: (0, 0))

    rows = e_pad // 128
    rb_pk, ysh_pk = pl.pallas_call(
        _geom_kernel,
        grid=(1,),
        in_specs=[full(rows, 128)] * 5,
        out_specs=[full(rows, _RDIM * 128), full(rows, _NM * 128)],
        out_shape=[jax.ShapeDtypeStruct((rows, _RDIM * 128), _f32),
                   jax.ShapeDtypeStruct((rows, _NM * 128), _f32)],
        compiler_params=_tc_params(40, parallel=True),
    )(d2.reshape(rows, 128), sw2.reshape(rows, 128),
      vec2[:, 0].reshape(rows, 128), vec2[:, 1].reshape(rows, 128),
      vec2[:, 2].reshape(rows, 128))
    rb_e = (rb_pk.reshape(rows, _RDIM, 128).transpose(0, 2, 1)
            .reshape(e_pad, _RDIM))
    ysh_e = (ysh_pk.reshape(rows, _NM, 128).transpose(0, 2, 1)
             .reshape(e_pad, _NM))

    xi0, mi0p = pl.pallas_call(
        _node_pre_kernel,
        grid=(n_grid,),
        in_specs=[nbs(1), full(64, _DIM), full(1, _DIM), full(_DIM, _MSG)],
        out_specs=[nbs(_DIM), nbs(_G0W)],
        out_shape=[jax.ShapeDtypeStruct((N, _DIM), _f32),
                   jax.ShapeDtypeStruct((N, _G0W), _f32)],
        compiler_params=_tc_params(64, parallel=True),
    )(species.astype(jnp.int32).reshape(N, 1), wspec_p, bspec2, W_msg0)

    g0 = _sc_gather(mi0p, dst1)

    dens0 = pl.pallas_call(
        _seg0_kernel,
        grid=(e_pad // _EB,),
        in_specs=[ebs(1), ebs(_RDIM), ebs(_NM), ebs(_G0W),
                  full(_MSG, _NCD * _NM), full(_RDIM, _NCD * _NM),
                  full(_NM, _NCD * _NM)],
        out_specs=dspec,
        out_shape=jax.ShapeDtypeStruct((n_pad, _NCD * _NM), _f32),
        compiler_params=_tc_params(56),
    )(src2, rb_e, ysh_e, g0, repM, repR, repY)

    xi1, vi1, t1 = pl.pallas_call(
        _node_mid_kernel,
        grid=(n_grid,),
        in_specs=[nbs(_NCD * _NM), nbs(_DIM),
                  full(_NCD * _NM, _NCH * _NM), full(_NCD * _NM, _NCH * _NM),
                  full(_NCD * _NM, _NCH * _NM),
                  full(_NCH * _NM, _GX), full(_NCH * _NM, _GX),
                  full(_GX, _NCH * _NM), full(_GX, _NCH * _NM),
                  full(_DIM + _NCD + 2 * _NCH, _DIM), full(1, _DIM),
                  full(_DIM, _DIM), full(1, _DIM), full(_DIM, _MSG)],
        out_specs=[nbs(_DIM), nbs(_NCH * _NM), nbs(_G1W)],
        out_shape=[jax.ShapeDtypeStruct((N, _DIM), _f32),
                   jax.ShapeDtypeStruct((N, _NCH * _NM), _f32),
                   jax.ShapeDtypeStruct((N, _G1W), _f32)],
        compiler_params=_tc_params(64, parallel=True),
    )(dens0, xi0, wvi_bd, wdm00_bd, wdm01_bd, EV, EH, Ma0, Mb0,
      W_lat0a, bl0a, W_lat0b, bl0b, W_msg1)

    g1 = _sc_gather(t1, dst1)

    dens1 = pl.pallas_call(
        _seg1_kernel,
        grid=(e_pad // _EB,),
        in_specs=[ebs(1), ebs(_RDIM), ebs(_G1W),
                  full(_NCH * _NM, _NCD * _NM), full(_MSG, _NCD * _NM),
                  full(_RDIM, _NCD * _NM)],
        out_specs=dspec,
        out_shape=jax.ShapeDtypeStruct((n_pad, _NCD * _NM), _f32),
        compiler_params=_tc_params(56),
    )(src2, rb_e, g1, wrho_bd, repM, repR)

    xiF, viF = pl.pallas_call(
        _node_fin_kernel,
        grid=(n_grid,),
        in_specs=[nbs(_NCD * _NM), nbs(_NCD * _NM), nbs(_DIM),
                  nbs(_NCH * _NM),
                  full(_NCD * _NM, _NCH * _NM), full(_NCD * _NM, _NCH * _NM),
                  full(_NCH * _NM, _GX), full(_NCH * _NM, _GX),
                  full(_GX, _NCH * _NM), full(_GX, _NCH * _NM),
                  full(_DIM + _NCD + 2 * _NCH, _DIM), full(1, _DIM),
                  full(_DIM, _DIM), full(1, _DIM)],
        out_specs=[nbs(_DIM), nbs(_NCH * _NM)],
        out_shape=[jax.ShapeDtypeStruct((N, _DIM), _f32),
                   jax.ShapeDtypeStruct((N, _NCH * _NM), _f32)],
        compiler_params=_tc_params(64, parallel=True),
    )(dens0, dens1, xi1, vi1, wdm10_bd, wdm11_bd, EV, EH, Ma1, Mb1,
      W_lat1a, bl1a, W_lat1b, bl1b)

    Vi_out = viF.reshape(N, _NM, _NCH).transpose(0, 2, 1)
    return xiF, Vi_out
